# baseline (speedup 1.0000x reference)
import functools

import jax
import jax.numpy as jnp
from jax import lax
from jax.experimental import pallas as pl
from jax.experimental.pallas import tpu as pltpu
from jax.experimental.pallas import tpu_sc as plsc

SEQ_LEN = 8192
EMB_DIM = 1024
NUM_WORKERS = 32
B_PER_W = SEQ_LEN // NUM_WORKERS
CHUNKS = ((0, 120), (120, 120), (240, 16))
BUF_ROWS = (120, 120)


def _make_lookup():
  mesh = plsc.VectorSubcoreMesh(core_axis_name="c", subcore_axis_name="s")

  @functools.partial(
      pl.kernel,
      mesh=mesh,
      out_type=jax.ShapeDtypeStruct((SEQ_LEN, EMB_DIM), jnp.float32),
      scratch_types=[
          pltpu.VMEM((B_PER_W,), jnp.int32),
          pltpu.VMEM((BUF_ROWS[0], EMB_DIM), jnp.float32),
          pltpu.SemaphoreType.DMA,
          pltpu.SemaphoreType.DMA,
          pltpu.SemaphoreType.DMA,
      ],
  )
  def lookup(idx_hbm, table_hbm, out_hbm, idx_v, rows_a,
             gsem_a, gsem_b, isem):
    wid = lax.axis_index("s") * 2 + lax.axis_index("c")
    base = wid * B_PER_W
    bufs = (rows_a, rows_a)
    gsems = (gsem_a, gsem_b)
    first = CHUNKS[0][1]

    pltpu.sync_copy(idx_hbm.at[pl.ds(base, first)], idx_v.at[pl.ds(0, first)])

    def gather_copy(i):
      off, cnt = CHUNKS[i]
      buf = i % 2
      return pltpu.make_async_copy(
          table_hbm.at[idx_v.at[pl.ds(off, cnt)]],
          bufs[buf].at[pl.ds(0, cnt)],
          gsems[buf],
      )

    def write_back(i):
      off, cnt = CHUNKS[i]
      pltpu.sync_copy(bufs[i % 2].at[pl.ds(0, cnt)],
                      out_hbm.at[pl.ds(base + off, cnt)])

    gather_copy(0).start()
    rest = pltpu.make_async_copy(
        idx_hbm.at[pl.ds(base + first, B_PER_W - first)],
        idx_v.at[pl.ds(first, B_PER_W - first)],
        isem,
    )
    rest.start()
    rest.wait()

    for i in range(len(CHUNKS)):
      gather_copy(i).wait()
      write_back(i)
      if i + 1 < len(CHUNKS):
        gather_copy(i + 1).start()

  return lookup


_lookup = _make_lookup()


@jax.jit
def kernel(idxs, table):
  return _lookup(idxs.astype(jnp.int32), table)


# speedup vs baseline: 1.0268x; 1.0268x over previous
import functools

import jax
import jax.numpy as jnp
from jax import lax
from jax.experimental import pallas as pl
from jax.experimental.pallas import tpu as pltpu
from jax.experimental.pallas import tpu_sc as plsc

SEQ_LEN = 8192
EMB_DIM = 1024
NUM_WORKERS = 32
B_PER_W = SEQ_LEN // NUM_WORKERS
CHUNKS = ((0, 64), (64, 56), (120, 64), (184, 56), (240, 16))
BUF_ROWS = (64, 56)


def _make_lookup():
  mesh = plsc.VectorSubcoreMesh(core_axis_name="c", subcore_axis_name="s")

  @functools.partial(
      pl.kernel,
      mesh=mesh,
      out_type=jax.ShapeDtypeStruct((SEQ_LEN, EMB_DIM), jnp.float32),
      scratch_types=[
          pltpu.VMEM((B_PER_W,), jnp.int32),
          pltpu.VMEM((BUF_ROWS[0], EMB_DIM), jnp.float32),
          pltpu.VMEM((BUF_ROWS[1], EMB_DIM), jnp.float32),
          pltpu.SemaphoreType.DMA,
          pltpu.SemaphoreType.DMA,
          pltpu.SemaphoreType.DMA,
      ],
  )
  def lookup(idx_hbm, table_hbm, out_hbm, idx_v, rows_a, rows_b,
             gsem_a, gsem_b, isem):
    wid = lax.axis_index("c") * 16 + lax.axis_index("s")
    base = wid * B_PER_W
    bufs = (rows_a, rows_b)
    gsems = (gsem_a, gsem_b)
    first = CHUNKS[0][1]

    pltpu.sync_copy(idx_hbm.at[pl.ds(base, first)], idx_v.at[pl.ds(0, first)])

    def gather_copy(i):
      off, cnt = CHUNKS[i]
      buf = i % 2
      return pltpu.make_async_copy(
          table_hbm.at[idx_v.at[pl.ds(off, cnt)]],
          bufs[buf].at[pl.ds(0, cnt)],
          gsems[buf],
      )

    def write_back(i):
      off, cnt = CHUNKS[i]
      pltpu.sync_copy(bufs[i % 2].at[pl.ds(0, cnt)],
                      out_hbm.at[pl.ds(base + off, cnt)])

    gather_copy(0).start()
    rest = pltpu.make_async_copy(
        idx_hbm.at[pl.ds(base + first, B_PER_W - first)],
        idx_v.at[pl.ds(first, B_PER_W - first)],
        isem,
    )
    rest.start()
    rest.wait()

    for i in range(len(CHUNKS)):
      if i + 1 < len(CHUNKS):
        gather_copy(i + 1).start()
      gather_copy(i).wait()
      write_back(i)

  return lookup


_lookup = _make_lookup()


@jax.jit
def kernel(idxs, table):
  return _lookup(idxs.astype(jnp.int32), table)
